# baseline (speedup 1.0000x reference)
import functools

import jax
import jax.numpy as jnp
from jax import lax
from jax.experimental import pallas as pl
from jax.experimental.pallas import tpu as pltpu
from jax.experimental.pallas import tpu_sc as plsc

NC, NS, L = 2, 16, 16
NW = NC * NS

BATCH, FIELDS, D = 4096, 26, 32
VOC = 1000000
R = BATCH * FIELDS
RPW = R // NW
BRPW = BATCH // NW
OBB = 4
CH = OBB * FIELDS
NCH = BRPW // OBB
NG = CH // L + 1


def _rsqrt(v):
    i = lax.bitcast_convert_type(v, jnp.int32)
    y = lax.bitcast_convert_type(jnp.int32(0x5F3759DF) - (i >> 1), jnp.float32)
    for _ in range(3):
        y = y * (1.5 - 0.5 * v * y * y)
    return y


_mesh = plsc.VectorSubcoreMesh(core_axis_name="c", subcore_axis_name="s")


@functools.partial(
    pl.kernel,
    out_type=jax.ShapeDtypeStruct((BATCH, FIELDS * D), jnp.float32),
    mesh=_mesh,
    scratch_types=[
        pltpu.VMEM((1, RPW), jnp.int32),
        pltpu.VMEM((CH, 8, D), jnp.float32),
        pltpu.VMEM((2 * OBB, FIELDS * D), jnp.float32),
        pltpu.VMEM((D,), jnp.float32),
        pltpu.VMEM((D,), jnp.float32),
        pltpu.SemaphoreType.DMA,
    ],
)
def _embed_ln(x_hbm, table_hbm, gamma_hbm, beta_hbm, out_hbm,
              idx_v, slab_v, outbuf, gamma_v, beta_v, gsem):
    wid = lax.axis_index("s") * NC + lax.axis_index("c")

    pltpu.sync_copy(x_hbm.at[wid], idx_v)
    pltpu.sync_copy(gamma_hbm, gamma_v)
    pltpu.sync_copy(beta_hbm, beta_v)

    g_lo = gamma_v[pl.ds(0, L)]
    g_hi = gamma_v[pl.ds(L, L)]
    b_lo = beta_v[pl.ds(0, L)]
    b_hi = beta_v[pl.ds(L, L)]

    lane = lax.iota(jnp.int32, L)
    perms = [lane ^ (1 << k) for k in range(4)]
    _dnums = lax.GatherDimensionNumbers(
        offset_dims=(), collapsed_slice_dims=(0,), start_index_map=(0,))

    def lane_perm(v, p):
        return lax.gather(v, p[:, None], _dnums, (1,),
                          mode=lax.GatherScatterMode.PROMISE_IN_BOUNDS)

    def allreduce_sum(v):
        for p in perms:
            v = v + lane_perm(v, p)
        return v

    def chunk_body(c, _):
        def fire_group(g, n):
            iv = idx_v[0, pl.ds(c * CH + g * L, L)]
            cps = []
            for j in range(n):
                q8 = pl.multiple_of((iv[j] >> 3) * 8, 8)
                cps.append(pltpu.make_async_copy(
                    table_hbm.at[pl.ds(q8, 8)], slab_v.at[g * L + j], gsem))
            for cp in cps:
                cp.start()
            for cp in cps:
                cp.wait()

        def fire_full(g, _):
            fire_group(g, L)
            return 0

        lax.fori_loop(0, NG - 1, fire_full, 0)
        ivt = idx_v[0, pl.ds(c * CH + CH - L, L)]
        tps = []
        for j in range(8, L):
            q8 = pl.multiple_of((ivt[j] >> 3) * 8, 8)
            tps.append(pltpu.make_async_copy(
                table_hbm.at[pl.ds(q8, 8)], slab_v.at[CH - L + j], gsem))
        for cp in tps:
            cp.start()
        for cp in tps:
            cp.wait()

        def ln_rows(ob, _):
            base = c * CH + ob * FIELDS
            iva = idx_v[0, pl.ds(base, L)]
            ivb = idx_v[0, pl.ds(base + 10, L)]
            for f in range(FIELDS):
                slot = ob * FIELDS + f
                sub = (iva[f] if f < L else ivb[f - 10]) & 7
                a = slab_v[slot, sub, pl.ds(0, L)]
                bb = slab_v[slot, sub, pl.ds(L, L)]
                total = allreduce_sum(a + bb)
                total2 = allreduce_sum(a * a + bb * bb)
                mean = total * (1.0 / D)
                var = total2 * (1.0 / D) - mean * mean
                rstd = _rsqrt(var + 1e-5)
                orow = (c & 1) * OBB + ob
                outbuf[orow, pl.ds(f * D, L)] = (a - mean) * rstd * g_lo + b_lo
                outbuf[orow, pl.ds(f * D + L, L)] = (bb - mean) * rstd * g_hi + b_hi
            return 0

        lax.fori_loop(0, OBB, ln_rows, 0)

        @pl.when(c & 1 == 1)
        def _():
            row0 = pl.multiple_of(wid * BRPW + (c - 1) * OBB, 8)
            pltpu.sync_copy(outbuf, out_hbm.at[pl.ds(row0, 2 * OBB)])
        return 0

    lax.fori_loop(0, NCH, chunk_body, 0)


def kernel(x, table, gamma, beta):
    x3d = x.reshape(NW, 1, RPW)
    return _embed_ln(x3d, table, gamma, beta)
